# Initial kernel scaffold; baseline (speedup 1.0000x reference)
import jax
import jax.numpy as jnp
from jax.experimental import pallas as pl


def kernel(x, center, ptr, trigger):
    raise NotImplementedError("write your pallas kernel here")


# speedup vs baseline: 7.3709x; 7.3709x over previous
import functools

import jax
import jax.numpy as jnp
from jax import lax
from jax.experimental import pallas as pl
from jax.experimental.pallas import tpu as pltpu
from jax.experimental.pallas import tpu_sc as plsc

_N = 262144
_D = 128
_B = 4096
_CLAMP_LO = 32
_CLAMP_HI = 96

_MBLK = 512
_MROWS = 8


def _prep_body(c_col, p_col, c_row, p_row, trig, cp_out, add_out):
    a = c_col[...] + p_col[...]
    b = c_row[...] + p_row[...]
    acc = jnp.zeros((_MBLK, 1), dtype=jnp.int32)
    for i in range(_MROWS):
        eq = (a == b[i : i + 1, :]).astype(jnp.int32)
        acc = acc + jnp.sum(eq, axis=1, keepdims=True)
    cp_out[...] = a
    add_out[...] = acc.astype(jnp.float32) * trig[...]


_prep_call = pl.pallas_call(
    _prep_body,
    grid=(_B // _MBLK,),
    in_specs=[
        pl.BlockSpec((_MBLK, 1), lambda i: (i, 0)),
        pl.BlockSpec((_MBLK, 1), lambda i: (i, 0)),
        pl.BlockSpec((_MROWS, _B // _MROWS), lambda i: (0, 0)),
        pl.BlockSpec((_MROWS, _B // _MROWS), lambda i: (0, 0)),
        pl.BlockSpec((1, _D), lambda i: (0, 0)),
    ],
    out_specs=[
        pl.BlockSpec((_MBLK, 1), lambda i: (i, 0)),
        pl.BlockSpec((_MBLK, _D), lambda i: (i, 0)),
    ],
    out_shape=[
        jax.ShapeDtypeStruct((_B, 1), jnp.int32),
        jax.ShapeDtypeStruct((_B, _D), jnp.float32),
    ],
)

_CBLK = 4096


def _clamp_body(x_ref, o_ref):
    xv = x_ref[...]
    col = lax.broadcasted_iota(jnp.int32, xv.shape, 1)
    mid = (col >= _CLAMP_LO) & (col < _CLAMP_HI)
    o_ref[...] = jnp.where(mid, jnp.minimum(xv, 1.0), xv)


_clamp_call = pl.pallas_call(
    _clamp_body,
    grid=(_N // _CBLK,),
    in_specs=[pl.BlockSpec((_CBLK, _D), lambda i: (i, 0))],
    out_specs=pl.BlockSpec((_CBLK, _D), lambda i: (i, 0)),
    out_shape=jax.ShapeDtypeStruct((_N, _D), jnp.float32),
)

_NC = 2
_NS = 16
_NW = _NC * _NS
_RPW = _B // _NW
_L = 16


def _sc_scatter_body(x_hbm, cp_hbm, add_hbm, out_ref, idx_v, rows_v, add_v, sem):
    wid = lax.axis_index("s") * _NC + lax.axis_index("c")
    base = wid * _RPW
    pltpu.sync_copy(cp_hbm.at[pl.ds(base, _RPW)], idx_v)
    pltpu.sync_copy(add_hbm.at[pl.ds(base, _RPW)], add_v)
    pltpu.async_copy(x_hbm.at[idx_v], rows_v, sem).wait()

    @pl.loop(0, _RPW)
    def _row(j):
        for c in range(_D // _L):
            v = rows_v[j, pl.ds(c * _L, _L)] + add_v[j, pl.ds(c * _L, _L)]
            if _CLAMP_LO <= c * _L < _CLAMP_HI:
                v = jnp.minimum(v, 1.0)
            rows_v[j, pl.ds(c * _L, _L)] = v

    pltpu.async_copy(rows_v, out_ref.at[idx_v], sem).wait()


_sc_scatter = pl.kernel(
    _sc_scatter_body,
    out_type=(),
    mesh=plsc.VectorSubcoreMesh(
        core_axis_name="c", subcore_axis_name="s", num_cores=_NC, num_subcores=_NS
    ),
    scratch_types=[
        pltpu.VMEM((_RPW,), jnp.int32),
        pltpu.VMEM((_RPW, _D), jnp.float32),
        pltpu.VMEM((_RPW, _D), jnp.float32),
        pltpu.SemaphoreType.DMA,
    ],
)


def kernel(x, center, ptr, trigger):
    ptr_head = ptr[:_B]
    c_col = center.reshape(_B, 1)
    p_col = ptr_head.reshape(_B, 1)
    c_row = center.reshape(_MROWS, _B // _MROWS)
    p_row = ptr_head.reshape(_MROWS, _B // _MROWS)
    trig = trigger.reshape(1, _D)

    cp_col, addend = _prep_call(c_col, p_col, c_row, p_row, trig)
    cp = cp_col.reshape(_B)

    out1 = _clamp_call(x)

    ref = jax.new_ref(out1)
    _sc_scatter(x, cp, addend, ref)
    return jax.freeze(ref)


# speedup vs baseline: 7.7804x; 1.0556x over previous
import functools

import jax
import jax.numpy as jnp
from jax import lax
from jax.experimental import pallas as pl
from jax.experimental.pallas import tpu as pltpu
from jax.experimental.pallas import tpu_sc as plsc

_N = 262144
_D = 128
_B = 4096
_CLAMP_LO = 32
_CLAMP_HI = 96

_MBLK = 512
_MROWS = 8


def _prep_body(c_col, p_col, c_row, p_row, trig, cp_out, add_out):
    a = c_col[...] + p_col[...]
    b = c_row[...] + p_row[...]
    acc = jnp.zeros((_MBLK, 1), dtype=jnp.int32)
    for i in range(_MROWS):
        eq = (a == b[i : i + 1, :]).astype(jnp.int32)
        acc = acc + jnp.sum(eq, axis=1, keepdims=True)
    cp_out[...] = a
    add_out[...] = acc.astype(jnp.float32) * trig[...]


_prep_call = pl.pallas_call(
    _prep_body,
    grid=(_B // _MBLK,),
    in_specs=[
        pl.BlockSpec((_MBLK, 1), lambda i: (i, 0)),
        pl.BlockSpec((_MBLK, 1), lambda i: (i, 0)),
        pl.BlockSpec((_MROWS, _B // _MROWS), lambda i: (0, 0)),
        pl.BlockSpec((_MROWS, _B // _MROWS), lambda i: (0, 0)),
        pl.BlockSpec((1, _D), lambda i: (0, 0)),
    ],
    out_specs=[
        pl.BlockSpec((_MBLK, 1), lambda i: (i, 0)),
        pl.BlockSpec((_MBLK, _D), lambda i: (i, 0)),
    ],
    out_shape=[
        jax.ShapeDtypeStruct((_B, 1), jnp.int32),
        jax.ShapeDtypeStruct((_B, _D), jnp.float32),
    ],
)

_CBLK = 8192


def _clamp_body(x_ref, o_ref):
    xv = x_ref[...]
    col = lax.broadcasted_iota(jnp.int32, xv.shape, 1)
    mid = (col >= _CLAMP_LO) & (col < _CLAMP_HI)
    o_ref[...] = jnp.where(mid, jnp.minimum(xv, 1.0), xv)


_clamp_call = pl.pallas_call(
    _clamp_body,
    grid=(_N // _CBLK,),
    in_specs=[pl.BlockSpec((_CBLK, _D), lambda i: (i, 0))],
    out_specs=pl.BlockSpec((_CBLK, _D), lambda i: (i, 0)),
    out_shape=jax.ShapeDtypeStruct((_N, _D), jnp.float32),
)

_NC = 2
_NS = 16
_NW = _NC * _NS
_RPW = _B // _NW
_L = 16


def _sc_scatter_body(x_hbm, cp_hbm, add_hbm, out_ref, idx_v, rows_v, add_v, sem):
    wid = lax.axis_index("s") * _NC + lax.axis_index("c")
    base = wid * _RPW
    pltpu.sync_copy(cp_hbm.at[pl.ds(base, _RPW)], idx_v)
    pltpu.sync_copy(add_hbm.at[pl.ds(base, _RPW)], add_v)
    pltpu.async_copy(x_hbm.at[idx_v], rows_v, sem).wait()

    @pl.loop(0, _RPW)
    def _row(j):
        for c in range(_D // _L):
            v = rows_v[j, pl.ds(c * _L, _L)] + add_v[j, pl.ds(c * _L, _L)]
            if _CLAMP_LO <= c * _L < _CLAMP_HI:
                v = jnp.minimum(v, 1.0)
            rows_v[j, pl.ds(c * _L, _L)] = v

    pltpu.async_copy(rows_v, out_ref.at[idx_v], sem).wait()


_sc_scatter = pl.kernel(
    _sc_scatter_body,
    out_type=(),
    mesh=plsc.VectorSubcoreMesh(
        core_axis_name="c", subcore_axis_name="s", num_cores=_NC, num_subcores=_NS
    ),
    scratch_types=[
        pltpu.VMEM((_RPW,), jnp.int32),
        pltpu.VMEM((_RPW, _D), jnp.float32),
        pltpu.VMEM((_RPW, _D), jnp.float32),
        pltpu.SemaphoreType.DMA,
    ],
)


def kernel(x, center, ptr, trigger):
    ptr_head = ptr[:_B]
    c_col = center.reshape(_B, 1)
    p_col = ptr_head.reshape(_B, 1)
    c_row = center.reshape(_MROWS, _B // _MROWS)
    p_row = ptr_head.reshape(_MROWS, _B // _MROWS)
    trig = trigger.reshape(1, _D)

    cp_col, addend = _prep_call(c_col, p_col, c_row, p_row, trig)
    cp = cp_col.reshape(_B)

    out1 = _clamp_call(x)

    ref = jax.new_ref(out1)
    _sc_scatter(x, cp, addend, ref)
    return jax.freeze(ref)


# speedup vs baseline: 7.8876x; 1.0138x over previous
import functools

import jax
import jax.numpy as jnp
from jax import lax
from jax.experimental import pallas as pl
from jax.experimental.pallas import tpu as pltpu
from jax.experimental.pallas import tpu_sc as plsc

_N = 262144
_D = 128
_B = 4096
_CLAMP_LO = 32
_CLAMP_HI = 96

_MBLK = 512
_MROWS = 8


def _prep_body(c_col, p_col, c_row, p_row, trig, cp_out, add_out):
    a = c_col[...] + p_col[...]
    b = c_row[...] + p_row[...]
    acc = jnp.zeros((_MBLK, 1), dtype=jnp.int32)
    for i in range(_MROWS):
        eq = (a == b[i : i + 1, :]).astype(jnp.int32)
        acc = acc + jnp.sum(eq, axis=1, keepdims=True)
    cp_out[...] = a
    add_out[...] = acc.astype(jnp.float32) * trig[...]


_prep_call = pl.pallas_call(
    _prep_body,
    grid=(_B // _MBLK,),
    in_specs=[
        pl.BlockSpec((_MBLK, 1), lambda i: (i, 0)),
        pl.BlockSpec((_MBLK, 1), lambda i: (i, 0)),
        pl.BlockSpec((_MROWS, _B // _MROWS), lambda i: (0, 0)),
        pl.BlockSpec((_MROWS, _B // _MROWS), lambda i: (0, 0)),
        pl.BlockSpec((1, _D), lambda i: (0, 0)),
    ],
    out_specs=[
        pl.BlockSpec((_MBLK, 1), lambda i: (i, 0)),
        pl.BlockSpec((_MBLK, _D), lambda i: (i, 0)),
    ],
    out_shape=[
        jax.ShapeDtypeStruct((_B, 1), jnp.int32),
        jax.ShapeDtypeStruct((_B, _D), jnp.float32),
    ],
)

_CBLK = 16384


def _clamp_body(x_ref, o_ref):
    xv = x_ref[...]
    col = lax.broadcasted_iota(jnp.int32, xv.shape, 1)
    mid = (col >= _CLAMP_LO) & (col < _CLAMP_HI)
    o_ref[...] = jnp.where(mid, jnp.minimum(xv, 1.0), xv)


_clamp_call = pl.pallas_call(
    _clamp_body,
    grid=(_N // _CBLK,),
    in_specs=[pl.BlockSpec((_CBLK, _D), lambda i: (i, 0))],
    out_specs=pl.BlockSpec((_CBLK, _D), lambda i: (i, 0)),
    out_shape=jax.ShapeDtypeStruct((_N, _D), jnp.float32),
)

_NC = 2
_NS = 16
_NW = _NC * _NS
_RPW = _B // _NW
_L = 16


def _sc_scatter_body(x_hbm, cp_hbm, add_hbm, out_ref, idx_v, rows_v, add_v, sem):
    wid = lax.axis_index("s") * _NC + lax.axis_index("c")
    base = wid * _RPW
    pltpu.sync_copy(cp_hbm.at[pl.ds(base, _RPW)], idx_v)
    pltpu.sync_copy(add_hbm.at[pl.ds(base, _RPW)], add_v)
    pltpu.async_copy(x_hbm.at[idx_v], rows_v, sem).wait()

    @pl.loop(0, _RPW)
    def _row(j):
        for c in range(_D // _L):
            v = rows_v[j, pl.ds(c * _L, _L)] + add_v[j, pl.ds(c * _L, _L)]
            if _CLAMP_LO <= c * _L < _CLAMP_HI:
                v = jnp.minimum(v, 1.0)
            rows_v[j, pl.ds(c * _L, _L)] = v

    pltpu.async_copy(rows_v, out_ref.at[idx_v], sem).wait()


_sc_scatter = pl.kernel(
    _sc_scatter_body,
    out_type=(),
    mesh=plsc.VectorSubcoreMesh(
        core_axis_name="c", subcore_axis_name="s", num_cores=_NC, num_subcores=_NS
    ),
    scratch_types=[
        pltpu.VMEM((_RPW,), jnp.int32),
        pltpu.VMEM((_RPW, _D), jnp.float32),
        pltpu.VMEM((_RPW, _D), jnp.float32),
        pltpu.SemaphoreType.DMA,
    ],
)


def kernel(x, center, ptr, trigger):
    ptr_head = ptr[:_B]
    c_col = center.reshape(_B, 1)
    p_col = ptr_head.reshape(_B, 1)
    c_row = center.reshape(_MROWS, _B // _MROWS)
    p_row = ptr_head.reshape(_MROWS, _B // _MROWS)
    trig = trigger.reshape(1, _D)

    cp_col, addend = _prep_call(c_col, p_col, c_row, p_row, trig)
    cp = cp_col.reshape(_B)

    out1 = _clamp_call(x)

    ref = jax.new_ref(out1)
    _sc_scatter(x, cp, addend, ref)
    return jax.freeze(ref)


# speedup vs baseline: 9.1156x; 1.1557x over previous
import functools

import jax
import jax.numpy as jnp
from jax import lax
from jax.experimental import pallas as pl
from jax.experimental.pallas import tpu as pltpu
from jax.experimental.pallas import tpu_sc as plsc

_N = 262144
_D = 128
_B = 4096
_CLAMP_LO = 32
_CLAMP_HI = 96

_MROWS = 8

_CBLK = 16384
_NPREP = _N // _CBLK
_MBLK = _B // _NPREP


def _clamp_prep_body(x_ref, c_row, p_row, trig, o_ref, add_out):
    i = pl.program_id(0)

    xv = x_ref[...]
    col = lax.broadcasted_iota(jnp.int32, xv.shape, 1)
    mid = (col >= _CLAMP_LO) & (col < _CLAMP_HI)
    o_ref[...] = jnp.where(mid, jnp.minimum(xv, 1.0), xv)

    b = c_row[...] + p_row[...]
    r, h = i // 2, (i % 2) * _MBLK
    a_row = (c_row[pl.ds(r, 1), pl.ds(h, _MBLK)]
             + p_row[pl.ds(r, 1), pl.ds(h, _MBLK)])
    a = jnp.transpose(a_row, (1, 0))
    acc = jnp.zeros((_MBLK, _B // _MROWS), dtype=jnp.int32)
    for k in range(_MROWS):
        acc = acc + (a == b[k : k + 1, :]).astype(jnp.int32)
    m = jnp.sum(acc, axis=1, keepdims=True)
    add_out[...] = m.astype(jnp.float32) * trig[...]


_full = lambda i: (0, 0)
_clamp_prep_call = pl.pallas_call(
    _clamp_prep_body,
    grid=(_N // _CBLK,),
    in_specs=[
        pl.BlockSpec((_CBLK, _D), lambda i: (i, 0)),
        pl.BlockSpec((_MROWS, _B // _MROWS), _full),
        pl.BlockSpec((_MROWS, _B // _MROWS), _full),
        pl.BlockSpec((1, _D), _full),
    ],
    out_specs=[
        pl.BlockSpec((_CBLK, _D), lambda i: (i, 0)),
        pl.BlockSpec((_MBLK, _D), lambda i: (i, 0)),
    ],
    out_shape=[
        jax.ShapeDtypeStruct((_N, _D), jnp.float32),
        jax.ShapeDtypeStruct((_B, _D), jnp.float32),
    ],
)

_NC = 2
_NS = 16
_NW = _NC * _NS
_RPW = _B // _NW
_L = 16


def _sc_scatter_body(
    x_hbm, ce_hbm, pt_hbm, add_hbm, out_ref, idx_v, tmp_v, rows_v, add_v, sem
):
    wid = lax.axis_index("s") * _NC + lax.axis_index("c")
    base = wid * _RPW
    pltpu.sync_copy(ce_hbm.at[pl.ds(base, _RPW)], idx_v)
    pltpu.sync_copy(pt_hbm.at[pl.ds(base, _RPW)], tmp_v)
    for k in range(_RPW // _L):
        s = pl.ds(k * _L, _L)
        idx_v[s] = idx_v[s] + tmp_v[s]
    pltpu.sync_copy(add_hbm.at[pl.ds(base, _RPW)], add_v)
    pltpu.async_copy(x_hbm.at[idx_v], rows_v, sem).wait()

    @pl.loop(0, _RPW)
    def _row(j):
        for c in range(_D // _L):
            v = rows_v[j, pl.ds(c * _L, _L)] + add_v[j, pl.ds(c * _L, _L)]
            if _CLAMP_LO <= c * _L < _CLAMP_HI:
                v = jnp.minimum(v, 1.0)
            rows_v[j, pl.ds(c * _L, _L)] = v

    pltpu.async_copy(rows_v, out_ref.at[idx_v], sem).wait()


_sc_scatter = pl.kernel(
    _sc_scatter_body,
    out_type=(),
    mesh=plsc.VectorSubcoreMesh(
        core_axis_name="c", subcore_axis_name="s", num_cores=_NC, num_subcores=_NS
    ),
    scratch_types=[
        pltpu.VMEM((_RPW,), jnp.int32),
        pltpu.VMEM((_RPW,), jnp.int32),
        pltpu.VMEM((_RPW, _D), jnp.float32),
        pltpu.VMEM((_RPW, _D), jnp.float32),
        pltpu.SemaphoreType.DMA,
    ],
)


def kernel(x, center, ptr, trigger):
    ptr_head = ptr[:_B]
    c_row = center.reshape(_MROWS, _B // _MROWS)
    p_row = ptr_head.reshape(_MROWS, _B // _MROWS)
    trig = trigger.reshape(1, _D)

    out1, addend = _clamp_prep_call(x, c_row, p_row, trig)

    ref = jax.new_ref(out1)
    _sc_scatter(x, center, ptr_head, addend, ref)
    return jax.freeze(ref)


# speedup vs baseline: 9.1175x; 1.0002x over previous
import functools

import jax
import jax.numpy as jnp
from jax import lax
from jax.experimental import pallas as pl
from jax.experimental.pallas import tpu as pltpu
from jax.experimental.pallas import tpu_sc as plsc

_N = 262144
_D = 128
_B = 4096
_CLAMP_LO = 32
_CLAMP_HI = 96

_MROWS = 8

_CBLK = 16384
_NPREP = _N // _CBLK
_MBLK = _B // _NPREP


def _clamp_prep_body(x_ref, c_row, p_row, trig, o_ref, add_out):
    i = pl.program_id(0)

    xv = x_ref[...]
    col = lax.broadcasted_iota(jnp.int32, xv.shape, 1)
    mid = (col >= _CLAMP_LO) & (col < _CLAMP_HI)
    o_ref[...] = jnp.where(mid, jnp.minimum(xv, 1.0), xv)

    b = c_row[...] + p_row[...]
    r, h = i // 2, (i % 2) * _MBLK
    a_row = (c_row[pl.ds(r, 1), pl.ds(h, _MBLK)]
             + p_row[pl.ds(r, 1), pl.ds(h, _MBLK)])
    a = jnp.transpose(a_row, (1, 0))
    acc = jnp.zeros((_MBLK, _B // _MROWS), dtype=jnp.int32)
    for k in range(_MROWS):
        acc = acc + (a == b[k : k + 1, :]).astype(jnp.int32)
    m = jnp.sum(acc, axis=1, keepdims=True)
    add_out[...] = m.astype(jnp.float32) * trig[...]


_full = lambda i: (0, 0)
_clamp_prep_call = pl.pallas_call(
    _clamp_prep_body,
    grid=(_N // _CBLK,),
    in_specs=[
        pl.BlockSpec((_CBLK, _D), lambda i: (i, 0)),
        pl.BlockSpec((_MROWS, _B // _MROWS), _full),
        pl.BlockSpec((_MROWS, _B // _MROWS), _full),
        pl.BlockSpec((1, _D), _full),
    ],
    out_specs=[
        pl.BlockSpec((_CBLK, _D), lambda i: (i, 0)),
        pl.BlockSpec((_MBLK, _D), lambda i: (i, 0)),
    ],
    out_shape=[
        jax.ShapeDtypeStruct((_N, _D), jnp.float32),
        jax.ShapeDtypeStruct((_B, _D), jnp.float32),
    ],
)

_NC = 2
_NS = 16
_NW = _NC * _NS
_RPW = _B // _NW
_L = 16


def _sc_scatter_body(
    x_hbm, ce_hbm, pt_hbm, add_hbm, out_ref, idx_v, tmp_v, rows_v, add_v, sem
):
    wid = lax.axis_index("s") * _NC + lax.axis_index("c")
    base = wid * _RPW
    pltpu.sync_copy(ce_hbm.at[pl.ds(base, _RPW)], idx_v)
    pltpu.sync_copy(pt_hbm.at[pl.ds(base, _RPW)], tmp_v)
    for k in range(_RPW // _L):
        s = pl.ds(k * _L, _L)
        idx_v[s] = idx_v[s] + tmp_v[s]
    pltpu.sync_copy(add_hbm.at[pl.ds(base, _RPW)], add_v)
    pltpu.async_copy(x_hbm.at[idx_v], rows_v, sem).wait()

    @plsc.parallel_loop(0, _RPW)
    def _row(j):
        for c in range(_D // _L):
            v = rows_v[j, pl.ds(c * _L, _L)] + add_v[j, pl.ds(c * _L, _L)]
            if _CLAMP_LO <= c * _L < _CLAMP_HI:
                v = jnp.minimum(v, 1.0)
            rows_v[j, pl.ds(c * _L, _L)] = v

    pltpu.async_copy(rows_v, out_ref.at[idx_v], sem).wait()


_sc_scatter = pl.kernel(
    _sc_scatter_body,
    out_type=(),
    mesh=plsc.VectorSubcoreMesh(
        core_axis_name="c", subcore_axis_name="s", num_cores=_NC, num_subcores=_NS
    ),
    scratch_types=[
        pltpu.VMEM((_RPW,), jnp.int32),
        pltpu.VMEM((_RPW,), jnp.int32),
        pltpu.VMEM((_RPW, _D), jnp.float32),
        pltpu.VMEM((_RPW, _D), jnp.float32),
        pltpu.SemaphoreType.DMA,
    ],
)


def kernel(x, center, ptr, trigger):
    ptr_head = ptr[:_B]
    c_row = center.reshape(_MROWS, _B // _MROWS)
    p_row = ptr_head.reshape(_MROWS, _B // _MROWS)
    trig = trigger.reshape(1, _D)

    out1, addend = _clamp_prep_call(x, c_row, p_row, trig)

    ref = jax.new_ref(out1)
    _sc_scatter(x, center, ptr_head, addend, ref)
    return jax.freeze(ref)
